# Initial kernel scaffold; baseline (speedup 1.0000x reference)
import functools

import jax
import jax.numpy as jnp
from jax import lax
from jax.experimental import pallas as pl
from jax.experimental.pallas import tpu as pltpu
from jax.experimental.pallas import tpu_sc as plsc

NC = 2
NS = 16
B = 80
RPC = 25


def _sc_scatter(n, e, h, featall, src2d, adj2d, dst2d, zacc, zdeg, ones_hbm):
    ept = e // NS
    rows_pt = ept // B
    chunks = rows_pt // RPC
    deg_rows_pt = rows_pt // NC
    deg_chunks = deg_rows_pt // RPC
    acc_slice = 2 * n // NS
    deg_slice = n // NS

    mesh = plsc.VectorSubcoreMesh(
        core_axis_name="c", subcore_axis_name="s", num_cores=NC, num_subcores=NS
    )

    @functools.partial(
        pl.kernel,
        out_type=[
            jax.ShapeDtypeStruct((2 * NC * n, h), jnp.float32),
            jax.ShapeDtypeStruct((NC * n, 16), jnp.float32),
        ],
        mesh=mesh,
        scratch_types=[
            pltpu.VMEM((RPC, B), jnp.int32),
            pltpu.VMEM((RPC, B), jnp.int32),
            pltpu.VMEM((RPC, B), jnp.int32),
            pltpu.VMEM((B, h), jnp.float32),
            pltpu.VMEM((B, 16), jnp.float32),
            pltpu.VMEM_SHARED((2 * n, h), jnp.float32),
            pltpu.VMEM_SHARED((n, 16), jnp.float32),
            pltpu.SemaphoreType.DMA,
        ],
    )
    def sc_kernel(feat_hbm, src_hbm, adj_hbm, dst_hbm, zacc_hbm, zdeg_hbm,
                  ones_hbm_ref, acc_out, deg_out,
                  srcbuf, adjbuf, degbuf, rows, ones_v, acc_sh, deg_sh, sem):
        c = lax.axis_index("c")
        s = lax.axis_index("s")

        pltpu.sync_copy(zacc_hbm.at[pl.ds(s * acc_slice, acc_slice)],
                        acc_sh.at[pl.ds(s * acc_slice, acc_slice)])
        pltpu.sync_copy(zdeg_hbm.at[pl.ds(s * deg_slice, deg_slice)],
                        deg_sh.at[pl.ds(s * deg_slice, deg_slice)])
        pltpu.sync_copy(ones_hbm_ref, ones_v)
        plsc.subcore_barrier()

        @pl.loop(0, chunks)
        def _(t):
            row0 = s * rows_pt + t * RPC
            pltpu.sync_copy(src_hbm.at[pl.ds(c * (e // B) + row0, RPC)], srcbuf)
            pltpu.sync_copy(adj_hbm.at[pl.ds(row0, RPC)], adjbuf)

            @pl.loop(0, RPC)
            def _(k):
                pltpu.async_copy(feat_hbm.at[srcbuf.at[k]], rows, sem).wait()
                pltpu.sync_copy(rows, acc_sh.at[adjbuf.at[k]], add=True)

        @pl.loop(0, deg_chunks)
        def _(t):
            row0 = s * rows_pt + c * deg_rows_pt + t * RPC
            pltpu.sync_copy(dst_hbm.at[pl.ds(row0, RPC)], degbuf)

            @pl.loop(0, RPC)
            def _(k):
                pltpu.sync_copy(ones_v, deg_sh.at[degbuf.at[k]], add=True)

        plsc.subcore_barrier()

        pltpu.sync_copy(acc_sh.at[pl.ds(s * acc_slice, acc_slice)],
                        acc_out.at[pl.ds(c * 2 * n + s * acc_slice, acc_slice)])
        pltpu.sync_copy(deg_sh.at[pl.ds(s * deg_slice, deg_slice)],
                        deg_out.at[pl.ds(c * n + s * deg_slice, deg_slice)])

    return sc_kernel(featall, src2d, adj2d, dst2d, zacc, zdeg, ones_hbm)


def _tc_body(h, pos0, pos1, neg0, neg1, deg0, deg1, x, wc, bb, o):
    deg = deg0[:, 0:1] + deg1[:, 0:1]
    denom = jnp.maximum(deg, 1.0)
    w = wc[...]
    dot = functools.partial(jnp.dot, precision=lax.Precision.HIGHEST,
                            preferred_element_type=jnp.float32)
    y = (dot(pos0[...], w[0:h]) + dot(pos1[...], w[h:2 * h])
         + dot(neg0[...], w[2 * h:3 * h]) + dot(neg1[...], w[3 * h:4 * h]))
    y = y / denom + dot(x[...], w[4 * h:6 * h]) + bb[...]
    n2 = jnp.sum(y * y, axis=1, keepdims=True)
    o[...] = y * lax.rsqrt(jnp.maximum(n2, 1e-24))


def _tc_combine(n, h, dout, acc, deg, feature, wc, bb):
    bn = 1000
    blocks = n // bn
    return pl.pallas_call(
        functools.partial(_tc_body, h),
        grid=(blocks,),
        in_specs=[
            pl.BlockSpec((bn, h), lambda j: (j, 0)),
            pl.BlockSpec((bn, h), lambda j: (j + 2 * blocks, 0)),
            pl.BlockSpec((bn, h), lambda j: (j + blocks, 0)),
            pl.BlockSpec((bn, h), lambda j: (j + 3 * blocks, 0)),
            pl.BlockSpec((bn, 16), lambda j: (j, 0)),
            pl.BlockSpec((bn, 16), lambda j: (j + blocks, 0)),
            pl.BlockSpec((bn, 2 * h), lambda j: (j, 0)),
            pl.BlockSpec((6 * h, dout), lambda j: (0, 0)),
            pl.BlockSpec((1, dout), lambda j: (0, 0)),
        ],
        out_specs=pl.BlockSpec((bn, dout), lambda j: (j, 0)),
        out_shape=jax.ShapeDtypeStruct((n, dout), jnp.float32),
    )(acc, acc, acc, acc, deg, deg, feature, wc, bb)


def kernel(feature, edge_index, edge_sign, W, b):
    n, d = feature.shape
    e = edge_index.shape[1]
    dout = W.shape[0]
    h = d // 2

    src = edge_index[0].astype(jnp.int32)
    dst = edge_index[1].astype(jnp.int32)
    adj = dst + jnp.where(edge_sign < 0, jnp.int32(n), jnp.int32(0))
    featall = jnp.concatenate([feature[:, :h], feature[:, h:]], axis=0)
    src2d = jnp.concatenate([src, src + n]).reshape(2 * e // B, B)
    adj2d = adj.reshape(e // B, B)
    dst2d = dst.reshape(e // B, B)
    zacc = jnp.zeros((2 * n, h), jnp.float32)
    zdeg = jnp.zeros((n, 16), jnp.float32)
    ones_hbm = jnp.ones((B, 16), jnp.float32)

    acc, deg = _sc_scatter(n, e, h, featall, src2d, adj2d, dst2d,
                           zacc, zdeg, ones_hbm)

    wc = W.T
    bb = b.reshape(1, dout)
    return _tc_combine(n, h, dout, acc, deg, feature, wc, bb)


# speedup vs baseline: 3.0601x; 3.0601x over previous
import dataclasses
import functools

import jax
import jax.numpy as jnp
from jax import lax
from jax.experimental import pallas as pl
from jax.experimental.pallas import tpu as pltpu
from jax.experimental.pallas import tpu_sc as plsc

NC = 2
NS = 16
NT = NC * NS
BB = 128
LCAP = 10240
SEGW = 3392
NREL = 3456
DUMPR = 3408
HSZ = NREL * 16
ACHUNK = 2000


def _mesh():
    return plsc.VectorSubcoreMesh(
        core_axis_name="c", subcore_axis_name="s", num_cores=NC, num_subcores=NS
    )


def _compiler_params():
    cp = pltpu.CompilerParams()
    if "needs_layout_passes" in pltpu.CompilerParams.__dataclass_fields__:
        cp = dataclasses.replace(cp, needs_layout_passes=False)
    return cp


def _sc_partition(e, src, dst, sgn):
    ept = e // NT
    achunks = ept // ACHUNK

    @functools.partial(
        pl.kernel,
        out_type=[
            jax.ShapeDtypeStruct((6 * NT * LCAP,), jnp.int32),
            jax.ShapeDtypeStruct((6 * NT * LCAP,), jnp.int32),
            jax.ShapeDtypeStruct((6 * NT * 16,), jnp.int32),
        ],
        mesh=_mesh(),
        scratch_types=[
            pltpu.VMEM((ACHUNK,), jnp.int32),
            pltpu.VMEM((ACHUNK,), jnp.int32),
            pltpu.VMEM((ACHUNK,), jnp.float32),
            pltpu.VMEM((LCAP,), jnp.int32),
            pltpu.VMEM((LCAP,), jnp.int32),
            pltpu.VMEM((LCAP,), jnp.int32),
            pltpu.VMEM((LCAP,), jnp.int32),
            pltpu.VMEM((LCAP,), jnp.int32),
            pltpu.VMEM((LCAP,), jnp.int32),
            pltpu.VMEM((LCAP,), jnp.int32),
            pltpu.VMEM((LCAP,), jnp.int32),
            pltpu.VMEM((LCAP,), jnp.int32),
            pltpu.VMEM((LCAP,), jnp.int32),
            pltpu.VMEM((LCAP,), jnp.int32),
            pltpu.VMEM((LCAP,), jnp.int32),
            pltpu.VMEM((16,), jnp.int32),
        ],
        compiler_params=_compiler_params(),
    )
    def part_kernel(src_hbm, dst_hbm, sgn_hbm, lsrc_out, ldst_out, cnt_out,
                    ssrc, sdst, ssgn, s0, s1, s2, s3, s4, s5,
                    d0, d1, d2, d3, d4, d5, cntv):
        c = lax.axis_index("c")
        s = lax.axis_index("s")
        g = c * NS + s

        slists = (s0, s1, s2, s3, s4, s5)
        dlists = (d0, d1, d2, d3, d4, d5)

        zsrc = jnp.zeros((16,), jnp.int32)
        zdst = jnp.full((16,), DUMPR, jnp.int32)

        @pl.loop(0, LCAP, step=16)
        def _(i):
            for q in range(6):
                slists[q][pl.ds(i, 16)] = zsrc
                dlists[q][pl.ds(i, 16)] = zdst

        iota16 = lax.iota(jnp.int32, 16)
        tidx = (LCAP - 16) + iota16
        zero = jnp.int32(0)

        @pl.loop(0, achunks, init_carry=(zero,) * 6)
        def counts(t, carry):
            pltpu.sync_copy(src_hbm.at[pl.ds(g * ept + t * ACHUNK, ACHUNK)],
                            ssrc)
            pltpu.sync_copy(dst_hbm.at[pl.ds(g * ept + t * ACHUNK, ACHUNK)],
                            sdst)
            pltpu.sync_copy(sgn_hbm.at[pl.ds(g * ept + t * ACHUNK, ACHUNK)],
                            ssgn)

            @pl.loop(0, ACHUNK // 16, init_carry=carry)
            def inner(i, icarry):
                sv = ssrc[pl.ds(i * 16, 16)]
                dv = sdst[pl.ds(i * 16, 16)]
                mp = ssgn[pl.ds(i * 16, 16)] >= 0.0
                m1 = dv >= SEGW
                m2 = dv >= 2 * SEGW
                rel = dv - jnp.where(m2, 2 * SEGW, jnp.where(m1, SEGW, 0))
                third = m1.astype(jnp.int32) + m2.astype(jnp.int32)
                out = []
                for q in range(6):
                    mq = jnp.logical_and(
                        mp if q < 3 else jnp.logical_not(mp),
                        third == (q % 3))
                    mi = mq.astype(jnp.int32)
                    incl = plsc.cumsum(mi)
                    dest = jnp.where(mq, icarry[q] + incl - mi, tidx)
                    plsc.store_scatter(slists[q], [dest], sv)
                    plsc.store_scatter(dlists[q], [dest], rel)
                    out.append(icarry[q] + jnp.max(incl))
                return tuple(out)

            return inner

        for q in range(6):
            base = (q * NT + g) * LCAP
            pltpu.sync_copy(slists[q], lsrc_out.at[pl.ds(base, LCAP)])
            pltpu.sync_copy(dlists[q], ldst_out.at[pl.ds(base, LCAP)])
            cntv[...] = (jnp.full((16,), 0, jnp.int32)
                         + (counts[q] + (BB - 1)) // BB)
            pltpu.sync_copy(cntv, cnt_out.at[pl.ds((q * NT + g) * 16, 16)])

    return part_kernel(src, dst, sgn)


def _sc_accumulate(n, dfull, feat, lsrc, ldst, cnts):
    acc_slice = NREL // NS

    @functools.partial(
        pl.kernel,
        out_type=[
            jax.ShapeDtypeStruct((NC * 3 * NREL, dfull), jnp.float32),
            jax.ShapeDtypeStruct((NT * 3 * HSZ,), jnp.float32),
        ],
        mesh=_mesh(),
        scratch_types=[
            pltpu.VMEM((BB,), jnp.int32),
            pltpu.VMEM((BB,), jnp.int32),
            pltpu.VMEM((BB, dfull), jnp.float32),
            pltpu.VMEM((16,), jnp.int32),
            pltpu.VMEM((HSZ,), jnp.float32),
            pltpu.VMEM_SHARED((NREL, dfull), jnp.float32),
            pltpu.SemaphoreType.DMA,
        ],
        compiler_params=_compiler_params(),
    )
    def acc_kernel(feat_hbm, lsrc_hbm, ldst_hbm, cnt_hbm,
                   acc_out, deg_out,
                   srcv, dstv, rows, cntv, hist, acc_sh, sem):
        c = lax.axis_index("c")
        s = lax.axis_index("s")
        g = c * NS + s
        iota16 = lax.iota(jnp.int32, 16)
        zf16 = jnp.zeros((16,), jnp.float32)
        fone = jnp.ones((16,), jnp.float32)

        @pl.loop(0, 3)
        def _(h):
            @pl.loop(0, BB)
            def _(r):
                @pl.loop(0, dfull, step=16)
                def _(q):
                    rows[r, pl.ds(q, 16)] = zf16

            @pl.loop(0, acc_slice // BB)
            def _(i):
                pltpu.sync_copy(
                    rows, acc_sh.at[pl.ds(s * acc_slice + i * BB, BB)])

            rem = acc_slice % BB
            if rem:
                pltpu.sync_copy(
                    rows.at[pl.ds(0, rem)],
                    acc_sh.at[pl.ds(s * acc_slice + acc_slice - rem, rem)])

            @pl.loop(0, HSZ, step=16)
            def _(i):
                hist[pl.ds(i, 16)] = zf16

            plsc.subcore_barrier()

            @pl.loop(0, 2)
            def _(l):
                g2 = 2 * s + l
                lidx = (c * 3 + h) * NT + g2
                lbase = lidx * LCAP
                pltpu.sync_copy(cnt_hbm.at[pl.ds(lidx * 16, 16)], cntv)
                nb = jnp.max(cntv[...])

                @pl.loop(0, nb)
                def _(k):
                    pltpu.sync_copy(
                        lsrc_hbm.at[pl.ds(lbase + k * BB, BB)], srcv)
                    pltpu.sync_copy(
                        ldst_hbm.at[pl.ds(lbase + k * BB, BB)], dstv)
                    pltpu.async_copy(feat_hbm.at[srcv], rows, sem).wait()
                    pltpu.sync_copy(rows, acc_sh.at[dstv], add=True)

                    @pl.loop(0, BB // 16)
                    def _(j):
                        rel = dstv[pl.ds(j * 16, 16)]
                        plsc.addupdate_scatter(hist, [rel * 16 + iota16],
                                               fone)

            pltpu.sync_copy(hist, deg_out.at[pl.ds((g * 3 + h) * HSZ, HSZ)])
            plsc.subcore_barrier()

            pltpu.sync_copy(
                acc_sh.at[pl.ds(s * acc_slice, acc_slice)],
                acc_out.at[pl.ds((c * 3 + h) * NREL + s * acc_slice,
                                 acc_slice)])
            plsc.subcore_barrier()

    return acc_kernel(feat, lsrc, ldst, cnts)


def _tc_body(d, pos, neg, hist, x, wc, bb, o):
    deg = jnp.sum(hist[...], axis=(0, 1, 3))[:, None]
    denom = jnp.maximum(deg, 1.0)
    w = wc[...]
    dot = functools.partial(jnp.dot, precision=lax.Precision.HIGHEST,
                            preferred_element_type=jnp.float32)
    y = dot(pos[0][0], w[0:d]) + dot(neg[0][0], w[d:2 * d])
    y = y / denom + dot(x[...], w[2 * d:3 * d]) + bb[...]
    n2 = jnp.sum(y * y, axis=1, keepdims=True)
    o[...] = y * lax.rsqrt(jnp.maximum(n2, 1e-24))


def _tc_combine(n, d, dout, acc, deg, feature, wc, bb):
    bn = 424
    nb0 = SEGW // bn
    grid = ((n + bn - 1) // bn,)
    acc4 = acc.reshape(NC, 3, NREL, d)
    hist = deg.reshape(NT, 3, NREL, 16)
    return pl.pallas_call(
        functools.partial(_tc_body, d),
        grid=grid,
        in_specs=[
            pl.BlockSpec((1, 1, bn, d),
                         lambda j: (0, j // nb0, j % nb0, 0)),
            pl.BlockSpec((1, 1, bn, d),
                         lambda j: (1, j // nb0, j % nb0, 0)),
            pl.BlockSpec((NT, 1, bn, 16),
                         lambda j: (0, j // nb0, j % nb0, 0)),
            pl.BlockSpec((bn, d), lambda j: (j, 0)),
            pl.BlockSpec((3 * d, dout), lambda j: (0, 0)),
            pl.BlockSpec((1, dout), lambda j: (0, 0)),
        ],
        out_specs=pl.BlockSpec((bn, dout), lambda j: (j, 0)),
        out_shape=jax.ShapeDtypeStruct((n, dout), jnp.float32),
    )(acc4, acc4, hist, feature, wc, bb)


def kernel(feature, edge_index, edge_sign, W, b):
    n, d = feature.shape
    e = edge_index.shape[1]
    dout = W.shape[0]

    src = edge_index[0].astype(jnp.int32)
    dst = edge_index[1].astype(jnp.int32)

    lsrc, ldst, cnts = _sc_partition(e, src, dst, edge_sign)
    acc, deg = _sc_accumulate(n, d, feature, lsrc, ldst, cnts)

    wc = W.T
    bb = b.reshape(1, dout)
    return _tc_combine(n, d, dout, acc, deg, feature, wc, bb)
